# baseline (speedup 1.0000x reference)
import functools

import jax
import jax.numpy as jnp
from jax import lax
from jax.experimental import pallas as pl
from jax.experimental.pallas import tpu as pltpu
from jax.experimental.pallas import tpu_sc as plsc

_EMB = 32


@functools.cache
def _make_kernel(vocab, emb, batch):
    info = plsc.get_sparse_core_info()
    nc, ns = info.num_cores, info.num_subcores
    nw = nc * ns
    b_per_w = batch // nw

    mesh = plsc.VectorSubcoreMesh(core_axis_name="c", subcore_axis_name="s")

    @functools.partial(
        pl.kernel,
        mesh=mesh,
        compiler_params=pltpu.CompilerParams(use_tc_tiling_on_sc=False),
        out_type=(
            jax.ShapeDtypeStruct((batch, emb), jnp.float32),
            jax.ShapeDtypeStruct((batch, emb), jnp.float32),
        ),
        scratch_types=[
            pltpu.VMEM((b_per_w,), jnp.int32),
            pltpu.VMEM((b_per_w,), jnp.int32),
            pltpu.VMEM((b_per_w, emb), jnp.float32),
            pltpu.VMEM((b_per_w, emb), jnp.float32),
            pltpu.SemaphoreType.DMA,
            pltpu.SemaphoreType.DMA,
        ],
    )
    def gather_kernel(table, uids, iids, users_out, items_out,
                      uidx_v, iidx_v, urows_v, irows_v, usem, isem):
        wid = lax.axis_index("s") * nc + lax.axis_index("c")
        base = wid * b_per_w
        pltpu.sync_copy(uids.at[pl.ds(base, b_per_w)], uidx_v)
        ucopy = pltpu.async_copy(table.at[uidx_v], urows_v, usem)
        pltpu.sync_copy(iids.at[pl.ds(base, b_per_w)], iidx_v)
        icopy = pltpu.async_copy(table.at[iidx_v], irows_v, isem)
        ucopy.wait()
        pltpu.sync_copy(urows_v, users_out.at[pl.ds(base, b_per_w)])
        icopy.wait()
        pltpu.sync_copy(irows_v, items_out.at[pl.ds(base, b_per_w)])

    return gather_kernel


def kernel(embeddings, user_ids, item_ids):
    vocab, emb = embeddings.shape
    (batch,) = user_ids.shape
    fn = _make_kernel(vocab, emb, batch)
    return fn(embeddings, user_ids, item_ids)
